# Initial kernel scaffold; baseline (speedup 1.0000x reference)
import jax
import jax.numpy as jnp
from jax.experimental import pallas as pl


def kernel(x, edge_index, edge_attr, W_enc, b_enc, W1, b1, W2, b2, W_dec, b_dec):
    raise NotImplementedError("write your pallas kernel here")


# speedup vs baseline: 26.6563x; 26.6563x over previous
import functools

import jax
import jax.numpy as jnp
from jax import lax
from jax.experimental import pallas as pl
from jax.experimental.pallas import tpu as pltpu
from jax.experimental.pallas import tpu_sc as plsc

N = 10000
E = 320000
H = 16
NC = 2
NS = 16
NW = NC * NS
CW = 128
CHUNKS = 80
EPT = CHUNKS * CW
EP = NW * EPT
RP = 10240
SPT = RP // NS
DUMMY = N


def _zero_rows(rows_v):
    z = jnp.zeros((H,), jnp.float32)
    for i in range(CW):
        rows_v[i] = z


_sc_mesh = plsc.VectorSubcoreMesh(core_axis_name="c", subcore_axis_name="s")


@functools.partial(
    pl.kernel,
    out_type=jax.ShapeDtypeStruct((NC * RP, H), jnp.float32),
    mesh=_sc_mesh,
    scratch_types=[
        pltpu.VMEM((CHUNKS, CW), jnp.int32),
        pltpu.VMEM((CHUNKS, CW), jnp.int32),
        pltpu.VMEM((CW, H), jnp.float32),
        pltpu.VMEM_SHARED((RP, H), jnp.float32),
        pltpu.SemaphoreType.DMA,
    ],
    compiler_params=pltpu.CompilerParams(use_tc_tiling_on_sc=False),
)
def _layer_pass(ht_hbm, src_hbm, dst_hbm, out_hbm, src_v, dst_v, rows_v,
                acc_sh, sem):
    c = lax.axis_index("c")
    s = lax.axis_index("s")
    wid = c * NS + s
    pltpu.sync_copy(src_hbm.at[wid], src_v)
    pltpu.sync_copy(dst_hbm.at[wid], dst_v)
    _zero_rows(rows_v)
    for k in range(SPT // CW):
        pltpu.sync_copy(rows_v, acc_sh.at[pl.ds(s * SPT + k * CW, CW)])
    plsc.subcore_barrier()

    def body(j, _):
        pltpu.async_copy(ht_hbm.at[src_v.at[j]], rows_v, sem).wait()
        pltpu.sync_copy(rows_v, acc_sh.at[dst_v.at[j]], add=True)
        return ()

    lax.fori_loop(0, CHUNKS, body, (), unroll=False)
    plsc.subcore_barrier()
    pltpu.sync_copy(acc_sh.at[pl.ds(s * SPT, SPT)],
                    out_hbm.at[pl.ds(c * RP + s * SPT, SPT)])


@functools.partial(
    pl.kernel,
    out_type=jax.ShapeDtypeStruct((NC * RP,), jnp.float32),
    mesh=_sc_mesh,
    scratch_types=[
        pltpu.VMEM((CHUNKS, CW), jnp.int32),
        pltpu.VMEM((SPT,), jnp.float32),
        pltpu.VMEM_SHARED((RP,), jnp.float32),
    ],
    compiler_params=pltpu.CompilerParams(use_tc_tiling_on_sc=False),
)
def _degree_pass(dst_hbm, out_hbm, dst_v, buf_v, acc_sh):
    c = lax.axis_index("c")
    s = lax.axis_index("s")
    wid = c * NS + s
    pltpu.sync_copy(dst_hbm.at[wid], dst_v)
    z = jnp.zeros((H,), jnp.float32)
    for i in range(SPT // H):
        buf_v[pl.ds(i * H, H)] = z
    pltpu.sync_copy(buf_v, acc_sh.at[pl.ds(s * SPT, SPT)])
    plsc.subcore_barrier()
    one = jnp.ones((H,), jnp.float32)
    for i in range(CW // H):
        buf_v[pl.ds(i * H, H)] = one

    def body(j, _):
        pltpu.sync_copy(buf_v.at[pl.ds(0, CW)], acc_sh.at[dst_v.at[j]],
                        add=True)
        return ()

    lax.fori_loop(0, CHUNKS, body, (), unroll=False)
    plsc.subcore_barrier()
    pltpu.sync_copy(acc_sh.at[pl.ds(s * SPT, SPT)],
                    out_hbm.at[pl.ds(c * RP + s * SPT, SPT)])


def _matmul(a, b):
    return jnp.dot(a, b, preferred_element_type=jnp.float32,
                   precision=lax.Precision.HIGHEST)


def _enc_body(x_ref, w_ref, b_ref, d0_ref, d1_ref, ht_ref, dinv_ref):
    deg = d0_ref[...] + d1_ref[...] + 1.0
    dinv = lax.rsqrt(deg)
    h = jnp.maximum(_matmul(x_ref[...], w_ref[...]) + b_ref[...], 0.0)
    ht_ref[...] = h * dinv
    dinv_ref[...] = dinv


def _mid_body(s0_ref, s1_ref, ht_ref, dinv_ref, w_ref, b_ref, out_ref):
    dinv = dinv_ref[...]
    agg = (s0_ref[...] + s1_ref[...] + ht_ref[...]) * dinv
    h = jnp.maximum(_matmul(agg, w_ref[...]) + b_ref[...], 0.0)
    out_ref[...] = h * dinv


def _fin_body(s0_ref, s1_ref, ht_ref, dinv_ref, w_ref, b_ref, wd_ref, bd_ref,
              out_ref):
    agg = (s0_ref[...] + s1_ref[...] + ht_ref[...]) * dinv_ref[...]
    h = jnp.maximum(_matmul(agg, w_ref[...]) + b_ref[...], 0.0)
    out_ref[...] = _matmul(h, wd_ref[...]) + bd_ref[...]


_enc = pl.pallas_call(
    _enc_body,
    out_shape=[jax.ShapeDtypeStruct((N, H), jnp.float32),
               jax.ShapeDtypeStruct((N, 1), jnp.float32)],
)

_mid = pl.pallas_call(
    _mid_body,
    out_shape=jax.ShapeDtypeStruct((N, H), jnp.float32),
)

_fin = pl.pallas_call(
    _fin_body,
    out_shape=jax.ShapeDtypeStruct((N, 3), jnp.float32),
)


@jax.jit
def kernel(x, edge_index, edge_attr, W_enc, b_enc, W1, b1, W2, b2, W_dec,
           b_dec):
    src = edge_index[0].astype(jnp.int32)
    dst = edge_index[1].astype(jnp.int32)
    pad = EP - E
    src_p = jnp.concatenate([src, jnp.zeros((pad,), jnp.int32)])
    src_p = src_p.reshape(NW, CHUNKS, CW)
    dst_p = jnp.concatenate([dst, jnp.full((pad,), DUMMY, jnp.int32)])
    dst_p = dst_p.reshape(NW, CHUNKS, CW)

    deg = _degree_pass(dst_p)
    d0 = deg[:N].reshape(N, 1)
    d1 = deg[RP:RP + N].reshape(N, 1)

    ht1, dinv = _enc(x, W_enc, b_enc.reshape(1, H), d0, d1)

    s1 = _layer_pass(ht1, src_p, dst_p)
    ht2 = _mid(s1[:N], s1[RP:RP + N], ht1, dinv, W1, b1.reshape(1, H))

    s2 = _layer_pass(ht2, src_p, dst_p)
    out = _fin(s2[:N], s2[RP:RP + N], ht2, dinv, W2, b2.reshape(1, H),
               W_dec, b_dec.reshape(1, 3))
    return out
